# Initial kernel scaffold; baseline (speedup 1.0000x reference)
import jax
import jax.numpy as jnp
from jax.experimental import pallas as pl


def kernel(x, edge_index, edge_weights, W1, b1, W2, b2):
    raise NotImplementedError("write your pallas kernel here")


# speedup vs baseline: 6.0844x; 6.0844x over previous
import functools

import jax
import jax.numpy as jnp
from jax import lax
from jax.experimental import pallas as pl
from jax.experimental.pallas import tpu as pltpu
from jax.experimental.pallas import tpu_sc as plsc

N = 10000
E = 320000
D_IN = 128
D = 64

NC, NS, L = 2, 16, 16
NW = NC * NS
CH = 128
K = 80
EPW = CH * K
E_PAD = NW * EPW
NBLK = NW * K
N_PAD = 10112
ROWS_T = N_PAD // NS
DW = 16

_f32 = jnp.float32
_i32 = jnp.int32

_MESH = plsc.VectorSubcoreMesh(core_axis_name="c", subcore_axis_name="s")


@functools.partial(
    pl.kernel,
    out_type=jax.ShapeDtypeStruct((NC, N_PAD, D), _f32),
    mesh=_MESH,
    scratch_types=[
        pltpu.VMEM((K, CH), _i32),
        pltpu.VMEM((K, CH), _i32),
        pltpu.VMEM((CH, D), _f32),
        pltpu.VMEM_SHARED((N_PAD, D), _f32),
        pltpu.SemaphoreType.DMA,
    ],
    compiler_params=pltpu.CompilerParams(use_tc_tiling_on_sc=False, needs_layout_passes=False),
)
def _scatter_kernel(table, srcp, dstp, out, sidx, didx, rows_v, acc, sem):
    cid = lax.axis_index("c")
    sid = lax.axis_index("s")
    wid = cid * NS + sid

    pltpu.sync_copy(srcp.at[pl.ds(wid * K, K)], sidx)
    pltpu.sync_copy(dstp.at[pl.ds(wid * K, K)], didx)

    rows = pl.ds(sid * ROWS_T, ROWS_T)
    pltpu.sync_copy(table.at[rows], acc.at[rows])
    plsc.subcore_barrier()

    def body(j, carry):
        pltpu.async_copy(table.at[sidx.at[j]], rows_v, sem).wait()
        pltpu.sync_copy(rows_v, acc.at[didx.at[j]], add=True)
        return carry

    lax.fori_loop(0, K, body, 0)
    plsc.subcore_barrier()

    pltpu.sync_copy(acc.at[rows], out.at[cid, rows])


@functools.partial(
    pl.kernel,
    out_type=jax.ShapeDtypeStruct((NBLK, CH), _f32),
    mesh=_MESH,
    scratch_types=[
        pltpu.VMEM((K, CH), _i32),
        pltpu.VMEM((K, CH), _i32),
        pltpu.VMEM((CH, D), _f32),
        pltpu.VMEM((CH, D), _f32),
        pltpu.VMEM((K, CH), _f32),
        pltpu.SemaphoreType.DMA,
        pltpu.SemaphoreType.DMA,
    ],
    compiler_params=pltpu.CompilerParams(use_tc_tiling_on_sc=False, needs_layout_passes=False),
)
def _edge_dot_kernel(h2, srcp, dstp, out, sidx, didx, hs, hd, ob, sem0, sem1):
    cid = lax.axis_index("c")
    sid = lax.axis_index("s")
    wid = cid * NS + sid

    pltpu.sync_copy(srcp.at[pl.ds(wid * K, K)], sidx)
    pltpu.sync_copy(dstp.at[pl.ds(wid * K, K)], didx)

    lanes = lax.iota(_i32, L)

    def chunk(j, carry):
        cs = pltpu.async_copy(h2.at[sidx.at[j]], hs, sem0)
        cd = pltpu.async_copy(h2.at[didx.at[j]], hd, sem1)
        cs.wait()
        cd.wait()

        def group(g, c2):
            v = jnp.zeros((L,), _f32)
            for i in range(L):
                e = g * L + i
                a = hs[e, pl.ds(0, L)] * hd[e, pl.ds(0, L)]
                a += hs[e, pl.ds(L, L)] * hd[e, pl.ds(L, L)]
                a += hs[e, pl.ds(2 * L, L)] * hd[e, pl.ds(2 * L, L)]
                a += hs[e, pl.ds(3 * L, L)] * hd[e, pl.ds(3 * L, L)]
                v = jnp.where(lanes == i, jnp.sum(a), v)
            ob[j, pl.ds(g * L, L)] = v
            return c2

        lax.fori_loop(0, CH // L, group, 0)
        return carry

    lax.fori_loop(0, K, chunk, 0)
    pltpu.sync_copy(ob, out.at[pl.ds(wid * K, K)])


def _tc_stage_a(xp, W1, deg_s, deg_d):
    def body(x_ref, w_ref, ds_ref, dd_ref, s1_ref, ns_ref, nd_ref):
        out_deg = ds_ref[0] + ds_ref[1] - 1.0
        in_deg = dd_ref[0] + dd_ref[1] - 1.0
        ns = lax.rsqrt(out_deg[:, :1])
        nd = lax.rsqrt(in_deg[:, :1])
        ns_ref[...] = ns
        nd_ref[...] = nd
        s1_ref[...] = jnp.dot(x_ref[...], w_ref[...],
                              preferred_element_type=_f32) * ns

    return pl.pallas_call(
        body,
        out_shape=(
            jax.ShapeDtypeStruct((N_PAD, D), _f32),
            jax.ShapeDtypeStruct((N_PAD, 1), _f32),
            jax.ShapeDtypeStruct((N_PAD, 1), _f32),
        ),
    )(xp, W1, deg_s, deg_d)


def _tc_stage_b(part1, s1, ns, nd, W2, b1):
    def body(p_ref, s1_ref, ns_ref, nd_ref, w_ref, b_ref, s2_ref):
        agg = p_ref[0] + p_ref[1] - s1_ref[...]
        h1 = jnp.maximum(agg * nd_ref[...] + b_ref[...][None, :], 0.0)
        s2_ref[...] = jnp.dot(h1, w_ref[...],
                              preferred_element_type=_f32) * ns_ref[...]

    return pl.pallas_call(
        body,
        out_shape=jax.ShapeDtypeStruct((N_PAD, D), _f32),
    )(part1, s1, ns, nd, W2, b1)


def _tc_stage_c(part2, s2, nd, b2):
    def body(p_ref, s2_ref, nd_ref, b_ref, h2_ref, sl_ref):
        h2 = (p_ref[0] + p_ref[1] - s2_ref[...]) * nd_ref[...] + b_ref[...][None, :]
        h2_ref[...] = h2
        sl_ref[...] = jnp.sum(h2 * h2, axis=1, keepdims=True)

    return pl.pallas_call(
        body,
        out_shape=(
            jax.ShapeDtypeStruct((N_PAD, D), _f32),
            jax.ShapeDtypeStruct((N_PAD, 1), _f32),
        ),
    )(part2, s2, nd, b2)


def kernel(x, edge_index, edge_weights, W1, b1, W2, b2):
    del edge_weights

    pad = jnp.full((E_PAD - E,), N, dtype=_i32)
    srcp = jnp.concatenate([edge_index[0], pad]).reshape(NBLK, CH)
    dstp = jnp.concatenate([edge_index[1], pad]).reshape(NBLK, CH)
    xp = jnp.pad(x, ((0, N_PAD - N), (0, 0)))

    ones_table = jnp.ones((N_PAD, D), _f32)
    deg_s = _scatter_kernel(ones_table, srcp, srcp)
    deg_d = _scatter_kernel(ones_table, dstp, dstp)
    s1, ns, nd = _tc_stage_a(xp, W1, deg_s, deg_d)
    part1 = _scatter_kernel(s1, srcp, dstp)
    s2 = _tc_stage_b(part1, s1, ns, nd, W2, b1)
    part2 = _scatter_kernel(s2, srcp, dstp)
    h2, sl = _tc_stage_c(part2, s2, nd, b2)
    dots = _edge_dot_kernel(h2, srcp, dstp)

    return jnp.concatenate([dots.reshape(-1)[:E], sl[:N, 0]])


# speedup vs baseline: 8.0089x; 1.3163x over previous
import functools

import jax
import jax.numpy as jnp
from jax import lax
from jax.experimental import pallas as pl
from jax.experimental.pallas import tpu as pltpu
from jax.experimental.pallas import tpu_sc as plsc

N = 10000
E = 320000
D_IN = 128
D = 64

NC, NS, L = 2, 16, 16
NW = NC * NS
CH = 128
K = 80
EPW = CH * K
E_PAD = NW * EPW
NBLK = NW * K
N_PAD = 10112
ROWS_T = N_PAD // NS
DW = 16

_f32 = jnp.float32
_i32 = jnp.int32

_MESH = plsc.VectorSubcoreMesh(core_axis_name="c", subcore_axis_name="s")
_SC_PARAMS = pltpu.CompilerParams(use_tc_tiling_on_sc=False,
                                  needs_layout_passes=False)


@functools.partial(
    pl.kernel,
    out_type=(
        jax.ShapeDtypeStruct((NC, N_PAD, DW), _f32),
        jax.ShapeDtypeStruct((NC, N_PAD, DW), _f32),
    ),
    mesh=_MESH,
    scratch_types=[
        pltpu.VMEM((K, CH), _i32),
        pltpu.VMEM((K, CH), _i32),
        pltpu.VMEM((CH, DW), _f32),
        pltpu.VMEM((ROWS_T, DW), _f32),
        pltpu.VMEM_SHARED((N_PAD, DW), _f32),
        pltpu.VMEM_SHARED((N_PAD, DW), _f32),
    ],
    compiler_params=_SC_PARAMS,
)
def _deg_kernel(srcp, dstp, out_s, out_d, sidx, didx, ones_v, buf, acc_s, acc_d):
    cid = lax.axis_index("c")
    sid = lax.axis_index("s")
    wid = cid * NS + sid

    pltpu.sync_copy(srcp.at[pl.ds(wid * K, K)], sidx)
    pltpu.sync_copy(dstp.at[pl.ds(wid * K, K)], didx)

    def fill_ones(i, carry):
        ones_v[i, :] = jnp.ones((L,), _f32)
        return carry

    lax.fori_loop(0, CH, fill_ones, 0)

    def fill_zeros(i, carry):
        buf[i, :] = jnp.zeros((L,), _f32)
        return carry

    lax.fori_loop(0, ROWS_T, fill_zeros, 0)

    rows = pl.ds(sid * ROWS_T, ROWS_T)
    pltpu.sync_copy(buf, acc_s.at[rows])
    pltpu.sync_copy(buf, acc_d.at[rows])
    plsc.subcore_barrier()

    def body(j, carry):
        pltpu.sync_copy(ones_v, acc_s.at[sidx.at[j]], add=True)
        pltpu.sync_copy(ones_v, acc_d.at[didx.at[j]], add=True)
        return carry

    lax.fori_loop(0, K, body, 0)
    plsc.subcore_barrier()

    pltpu.sync_copy(acc_s.at[rows], out_s.at[cid, rows])
    pltpu.sync_copy(acc_d.at[rows], out_d.at[cid, rows])


@functools.partial(
    pl.kernel,
    out_type=jax.ShapeDtypeStruct((NC, N_PAD, D), _f32),
    mesh=_MESH,
    scratch_types=[
        pltpu.VMEM((K, CH), _i32),
        pltpu.VMEM((K, CH), _i32),
        pltpu.VMEM((2, CH, D), _f32),
        pltpu.VMEM_SHARED((N_PAD, D), _f32),
        pltpu.SemaphoreType.DMA,
        pltpu.SemaphoreType.DMA,
    ],
    compiler_params=_SC_PARAMS,
)
def _scatter_kernel(table, srcp, dstp, out, sidx, didx, rows_v, acc, sem0, sem1):
    cid = lax.axis_index("c")
    sid = lax.axis_index("s")
    wid = cid * NS + sid
    sems = (sem0, sem1)

    pltpu.sync_copy(srcp.at[pl.ds(wid * K, K)], sidx)
    pltpu.sync_copy(dstp.at[pl.ds(wid * K, K)], didx)

    rows = pl.ds(sid * ROWS_T, ROWS_T)
    pltpu.sync_copy(table.at[rows], acc.at[rows])
    plsc.subcore_barrier()

    for b in range(2):
        pltpu.async_copy(table.at[sidx.at[b]], rows_v.at[b], sems[b])

    @pl.loop(0, K, step=2)
    def _chunks(j):
        for b in range(2):
            jj = j + b
            pltpu.make_async_copy(table.at[sidx.at[jj]], rows_v.at[b],
                                  sems[b]).wait()
            pltpu.sync_copy(rows_v.at[b], acc.at[didx.at[jj]], add=True)

            @pl.when(jj + 2 < K)
            def _prefetch():
                pltpu.async_copy(table.at[sidx.at[jj + 2]], rows_v.at[b],
                                 sems[b])

    plsc.subcore_barrier()
    pltpu.sync_copy(acc.at[rows], out.at[cid, rows])


@functools.partial(
    pl.kernel,
    out_type=jax.ShapeDtypeStruct((NBLK, CH), _f32),
    mesh=_MESH,
    scratch_types=[
        pltpu.VMEM((K, CH), _i32),
        pltpu.VMEM((K, CH), _i32),
        pltpu.VMEM((2, CH, D), _f32),
        pltpu.VMEM((2, CH, D), _f32),
        pltpu.VMEM((K, CH), _f32),
        pltpu.SemaphoreType.DMA,
        pltpu.SemaphoreType.DMA,
        pltpu.SemaphoreType.DMA,
        pltpu.SemaphoreType.DMA,
    ],
    compiler_params=_SC_PARAMS,
)
def _edge_dot_kernel(h2, srcp, dstp, out, sidx, didx, hs, hd, ob,
                     ss0, ss1, sd0, sd1):
    cid = lax.axis_index("c")
    sid = lax.axis_index("s")
    wid = cid * NS + sid
    sems_s = (ss0, ss1)
    sems_d = (sd0, sd1)

    pltpu.sync_copy(srcp.at[pl.ds(wid * K, K)], sidx)
    pltpu.sync_copy(dstp.at[pl.ds(wid * K, K)], didx)

    lanes = lax.iota(_i32, L)

    for b in range(2):
        pltpu.async_copy(h2.at[sidx.at[b]], hs.at[b], sems_s[b])
        pltpu.async_copy(h2.at[didx.at[b]], hd.at[b], sems_d[b])

    @pl.loop(0, K, step=2)
    def _chunks(j):
        for b in range(2):
            jj = j + b
            pltpu.make_async_copy(h2.at[sidx.at[jj]], hs.at[b],
                                  sems_s[b]).wait()
            pltpu.make_async_copy(h2.at[didx.at[jj]], hd.at[b],
                                  sems_d[b]).wait()
            hsb = hs.at[b]
            hdb = hd.at[b]

            def group(g, c2):
                v = jnp.zeros((L,), _f32)
                for i in range(L):
                    e = g * L + i
                    a = hsb[e, pl.ds(0, L)] * hdb[e, pl.ds(0, L)]
                    a += hsb[e, pl.ds(L, L)] * hdb[e, pl.ds(L, L)]
                    a += hsb[e, pl.ds(2 * L, L)] * hdb[e, pl.ds(2 * L, L)]
                    a += hsb[e, pl.ds(3 * L, L)] * hdb[e, pl.ds(3 * L, L)]
                    v = jnp.where(lanes == i, jnp.sum(a), v)
                ob[jj, pl.ds(g * L, L)] = v
                return c2

            lax.fori_loop(0, CH // L, group, 0)

            @pl.when(jj + 2 < K)
            def _prefetch():
                pltpu.async_copy(h2.at[sidx.at[jj + 2]], hs.at[b], sems_s[b])
                pltpu.async_copy(h2.at[didx.at[jj + 2]], hd.at[b], sems_d[b])

    pltpu.sync_copy(ob, out.at[pl.ds(wid * K, K)])


def _tc_stage_a(xp, W1, deg_s, deg_d):
    def body(x_ref, w_ref, ds_ref, dd_ref, s1_ref, ns_ref, nd_ref):
        out_deg = ds_ref[0][:, :1] + ds_ref[1][:, :1] + 1.0
        in_deg = dd_ref[0][:, :1] + dd_ref[1][:, :1] + 1.0
        ns = lax.rsqrt(out_deg)
        nd = lax.rsqrt(in_deg)
        ns_ref[...] = ns
        nd_ref[...] = nd
        s1_ref[...] = jnp.dot(x_ref[...], w_ref[...],
                              preferred_element_type=_f32) * ns

    return pl.pallas_call(
        body,
        out_shape=(
            jax.ShapeDtypeStruct((N_PAD, D), _f32),
            jax.ShapeDtypeStruct((N_PAD, 1), _f32),
            jax.ShapeDtypeStruct((N_PAD, 1), _f32),
        ),
    )(xp, W1, deg_s, deg_d)


def _tc_stage_b(part1, s1, ns, nd, W2, b1):
    def body(p_ref, s1_ref, ns_ref, nd_ref, w_ref, b_ref, s2_ref):
        agg = p_ref[0] + p_ref[1] - s1_ref[...]
        h1 = jnp.maximum(agg * nd_ref[...] + b_ref[...][None, :], 0.0)
        s2_ref[...] = jnp.dot(h1, w_ref[...],
                              preferred_element_type=_f32) * ns_ref[...]

    return pl.pallas_call(
        body,
        out_shape=jax.ShapeDtypeStruct((N_PAD, D), _f32),
    )(part1, s1, ns, nd, W2, b1)


def _tc_stage_c(part2, s2, nd, b2):
    def body(p_ref, s2_ref, nd_ref, b_ref, h2_ref, sl_ref):
        h2 = (p_ref[0] + p_ref[1] - s2_ref[...]) * nd_ref[...] + b_ref[...][None, :]
        h2_ref[...] = h2
        sl_ref[...] = jnp.sum(h2 * h2, axis=1, keepdims=True)

    return pl.pallas_call(
        body,
        out_shape=(
            jax.ShapeDtypeStruct((N_PAD, D), _f32),
            jax.ShapeDtypeStruct((N_PAD, 1), _f32),
        ),
    )(part2, s2, nd, b2)


def kernel(x, edge_index, edge_weights, W1, b1, W2, b2):
    del edge_weights

    pad = jnp.full((E_PAD - E,), N, dtype=_i32)
    srcp = jnp.concatenate([edge_index[0], pad]).reshape(NBLK, CH)
    dstp = jnp.concatenate([edge_index[1], pad]).reshape(NBLK, CH)
    xp = jnp.pad(x, ((0, N_PAD - N), (0, 0)))

    deg_s, deg_d = _deg_kernel(srcp, dstp)
    s1, ns, nd = _tc_stage_a(xp, W1, deg_s, deg_d)
    part1 = _scatter_kernel(s1, srcp, dstp)
    s2 = _tc_stage_b(part1, s1, ns, nd, W2, b1)
    part2 = _scatter_kernel(s2, srcp, dstp)
    h2, sl = _tc_stage_c(part2, s2, nd, b2)
    dots = _edge_dot_kernel(h2, srcp, dstp)

    return jnp.concatenate([dots.reshape(-1)[:E], sl[:N, 0]])
